# Initial kernel scaffold; baseline (speedup 1.0000x reference)
import jax
import jax.numpy as jnp
from jax.experimental import pallas as pl


def kernel(mask, mesh_weight, index_map):
    raise NotImplementedError("write your pallas kernel here")


# speedup vs baseline: 21.5497x; 21.5497x over previous
import dataclasses
import functools

import jax
import jax.numpy as jnp
from jax import lax
from jax.experimental import pallas as pl
from jax.experimental.pallas import tpu as pltpu
from jax.experimental.pallas import tpu_sc as plsc

_NC = 2
_NS = 16
_L = 16
_NW = _NC * _NS
_NPIX = 8 * 512 * 512
_PW = _NPIX // _NW
_C = 4096
_NCHUNK = _PW // _C

_TPAD = 160
_MPAD = 768


def _sc_lookup(mask_flat, mesh_flat, im_pad):
    mesh = plsc.VectorSubcoreMesh(core_axis_name="c", subcore_axis_name="s")
    cp = pltpu.CompilerParams()
    if "needs_layout_passes" in pltpu.CompilerParams.__dataclass_fields__:
        cp = dataclasses.replace(cp, needs_layout_passes=False)

    @functools.partial(
        pl.kernel,
        compiler_params=cp,
        out_type=jax.ShapeDtypeStruct((_NPIX * 3,), jnp.float32),
        mesh=mesh,
        scratch_types=[
            pltpu.VMEM((_MPAD,), jnp.float32),
            pltpu.VMEM((_TPAD,), jnp.int32),
            pltpu.VMEM((3 * _TPAD,), jnp.float32),
            pltpu.VMEM((_C,), jnp.int32),
            pltpu.VMEM((3 * _C,), jnp.float32),
        ],
    )
    def k(mask_hbm, mesh_hbm, im_hbm, out_hbm, mesh_v, im_v, tab_v, mbuf, obuf):
        wid = lax.axis_index("s") * _NC + lax.axis_index("c")
        pltpu.sync_copy(mesh_hbm, mesh_v)
        pltpu.sync_copy(im_hbm, im_v)
        iota = lax.iota(jnp.int32, _L)
        pat3 = iota * 3

        for g in range(_TPAD // _L):
            imv = im_v[pl.ds(g * _L, _L)]
            j3 = imv * 3
            for d in range(3):
                v = plsc.load_gather(mesh_v, [j3 + d])
                tab_v[pl.ds(_TPAD * d + g * _L, _L)] = v

        base0 = wid * _PW

        @pl.loop(0, _NCHUNK)
        def _chunk(c):
            base = base0 + c * _C
            pltpu.sync_copy(mask_hbm.at[pl.ds(base, _C)], mbuf)

            @pl.loop(0, _C // _L)
            def _vec(i):
                j = mbuf[pl.ds(i * _L, _L)]
                o = pat3 + i * (3 * _L)
                for d in range(3):
                    v = plsc.load_gather(tab_v, [j + _TPAD * d])
                    plsc.store_scatter(obuf, [o + d], v)

            pltpu.sync_copy(obuf, out_hbm.at[pl.ds(base * 3, _C * 3)])

    return k(mask_flat, mesh_flat, im_pad)


def kernel(mask, mesh_weight, index_map):
    mask_flat = mask.reshape(-1)
    mesh_flat = jnp.pad(mesh_weight.reshape(-1), (0, _MPAD - mesh_weight.size))
    im_pad = jnp.pad(index_map, (0, _TPAD - index_map.shape[0]))
    out = _sc_lookup(mask_flat, mesh_flat, im_pad)
    return out.reshape(mask.shape + (3,))


# speedup vs baseline: 21.8902x; 1.0158x over previous
import dataclasses
import functools

import jax
import jax.numpy as jnp
from jax import lax
from jax.experimental import pallas as pl
from jax.experimental.pallas import tpu as pltpu
from jax.experimental.pallas import tpu_sc as plsc

_NC = 2
_NS = 16
_L = 16
_NW = _NC * _NS
_NPIX = 8 * 512 * 512
_PW = _NPIX // _NW
_C = 4096
_NCHUNK = _PW // _C
_U = 4

_TPAD = 160
_MPAD = 768


def _sc_lookup(mask_flat, mesh_flat, im_pad):
    mesh = plsc.VectorSubcoreMesh(core_axis_name="c", subcore_axis_name="s")
    cp = pltpu.CompilerParams()
    if "needs_layout_passes" in pltpu.CompilerParams.__dataclass_fields__:
        cp = dataclasses.replace(cp, needs_layout_passes=False)

    @functools.partial(
        pl.kernel,
        compiler_params=cp,
        out_type=jax.ShapeDtypeStruct((_NPIX * 3,), jnp.float32),
        mesh=mesh,
        scratch_types=[
            pltpu.VMEM((_MPAD,), jnp.float32),
            pltpu.VMEM((_TPAD,), jnp.int32),
            pltpu.VMEM((3 * _TPAD,), jnp.float32),
            pltpu.VMEM((_C,), jnp.int32),
            pltpu.VMEM((_C,), jnp.int32),
            pltpu.VMEM((3 * _C,), jnp.float32),
            pltpu.VMEM((3 * _C,), jnp.float32),
            pltpu.SemaphoreType.DMA,
            pltpu.SemaphoreType.DMA,
            pltpu.SemaphoreType.DMA,
            pltpu.SemaphoreType.DMA,
        ],
    )
    def k(mask_hbm, mesh_hbm, im_hbm, out_hbm, mesh_v, im_v, tab_v,
          mbuf0, mbuf1, obuf0, obuf1, sin0, sin1, sout0, sout1):
        wid = lax.axis_index("s") * _NC + lax.axis_index("c")
        base0 = wid * _PW
        mbuf = (mbuf0, mbuf1)
        obuf = (obuf0, obuf1)
        sin = (sin0, sin1)
        sout = (sout0, sout1)

        in_cp = [None] * _NCHUNK
        out_cp = [None] * _NCHUNK
        for c in range(2):
            in_cp[c] = pltpu.async_copy(
                mask_hbm.at[pl.ds(base0 + c * _C, _C)], mbuf[c], sin[c])

        pltpu.sync_copy(mesh_hbm, mesh_v)
        pltpu.sync_copy(im_hbm, im_v)
        iota = lax.iota(jnp.int32, _L)
        pat3 = iota * 3

        for g in range(_TPAD // _L):
            imv = im_v[pl.ds(g * _L, _L)]
            j3 = imv * 3
            for d in range(3):
                v = plsc.load_gather(mesh_v, [j3 + d])
                tab_v[pl.ds(_TPAD * d + g * _L, _L)] = v

        for c in range(_NCHUNK):
            b = c % 2
            if c >= 2:
                out_cp[c - 2].wait()
            in_cp[c].wait()

            mb = mbuf[b]
            ob = obuf[b]

            @pl.loop(0, _C // (_L * _U))
            def _vec(i, mb=mb, ob=ob):
                for u in range(_U):
                    j = mb[pl.ds(i * (_L * _U) + u * _L, _L)]
                    o = pat3 + (i * (3 * _L * _U) + u * (3 * _L))
                    for d in range(3):
                        v = plsc.load_gather(tab_v, [j + _TPAD * d])
                        plsc.store_scatter(ob, [o + d], v)

            out_cp[c] = pltpu.async_copy(
                ob, out_hbm.at[pl.ds((base0 + c * _C) * 3, _C * 3)], sout[b])
            if c + 2 < _NCHUNK:
                in_cp[c + 2] = pltpu.async_copy(
                    mask_hbm.at[pl.ds(base0 + (c + 2) * _C, _C)], mb, sin[b])

        out_cp[_NCHUNK - 2].wait()
        out_cp[_NCHUNK - 1].wait()

    return k(mask_flat, mesh_flat, im_pad)


def kernel(mask, mesh_weight, index_map):
    mask_flat = mask.reshape(-1)
    mesh_flat = jnp.pad(mesh_weight.reshape(-1), (0, _MPAD - mesh_weight.size))
    im_pad = jnp.pad(index_map, (0, _TPAD - index_map.shape[0]))
    out = _sc_lookup(mask_flat, mesh_flat, im_pad)
    return out.reshape(mask.shape + (3,))


# speedup vs baseline: 243.9580x; 11.1446x over previous
import dataclasses
import functools

import jax
import jax.numpy as jnp
from jax import lax
from jax.experimental import pallas as pl
from jax.experimental.pallas import tpu as pltpu
from jax.experimental.pallas import tpu_sc as plsc

_NC = 2
_NS = 16
_L = 16
_NW = _NC * _NS
_B, _H, _W = 8, 512, 512
_NPIX = _B * _H * _W
_PLANE = _H * _W
_PW = _NPIX // _NW
_WPB = _PLANE // _PW
_C = 4096
_NCHUNK = _PW // _C
_U = 4

_TPAD = 160
_MPAD = 768


def _sc_lookup(mask_flat, mesh_flat, im_pad):
    mesh = plsc.VectorSubcoreMesh(core_axis_name="c", subcore_axis_name="s")
    cp = pltpu.CompilerParams()
    if "needs_layout_passes" in pltpu.CompilerParams.__dataclass_fields__:
        cp = dataclasses.replace(cp, needs_layout_passes=False)

    @functools.partial(
        pl.kernel,
        compiler_params=cp,
        out_type=jax.ShapeDtypeStruct((_NPIX * 3,), jnp.float32),
        mesh=mesh,
        scratch_types=[
            pltpu.VMEM((_MPAD,), jnp.float32),
            pltpu.VMEM((_TPAD,), jnp.int32),
            pltpu.VMEM((3 * _TPAD,), jnp.float32),
            pltpu.VMEM((_C,), jnp.int32),
            pltpu.VMEM((_C,), jnp.int32),
            pltpu.VMEM((3 * _C,), jnp.float32),
            pltpu.VMEM((3 * _C,), jnp.float32),
            pltpu.SemaphoreType.DMA,
            pltpu.SemaphoreType.DMA,
            pltpu.SemaphoreType.DMA,
            pltpu.SemaphoreType.DMA,
        ],
    )
    def k(mask_hbm, mesh_hbm, im_hbm, out_hbm, mesh_v, im_v, tab_v,
          mbuf0, mbuf1, obuf0, obuf1, sin0, sin1, sout0, sout1):
        wid = lax.axis_index("s") * _NC + lax.axis_index("c")
        base0 = wid * _PW
        b_img = wid // _WPB
        po = (wid % _WPB) * _PW
        mbuf = (mbuf0, mbuf1)
        obuf = (obuf0, obuf1)
        sin = (sin0, sin1)
        sout = (sout0, sout1)

        in_cp = [None] * _NCHUNK
        out_cp = [None] * _NCHUNK
        for c in range(2):
            in_cp[c] = pltpu.async_copy(
                mask_hbm.at[pl.ds(base0 + c * _C, _C)], mbuf[c], sin[c])

        pltpu.sync_copy(mesh_hbm, mesh_v)
        pltpu.sync_copy(im_hbm, im_v)

        for g in range(_TPAD // _L):
            imv = im_v[pl.ds(g * _L, _L)]
            j3 = imv * 3
            for d in range(3):
                v = plsc.load_gather(mesh_v, [j3 + d])
                tab_v[pl.ds(_TPAD * d + g * _L, _L)] = v

        for c in range(_NCHUNK):
            b = c % 2
            if c >= 2:
                for cp_ in out_cp[c - 2]:
                    cp_.wait()
            in_cp[c].wait()

            mb = mbuf[b]
            ob = obuf[b]

            @pl.loop(0, _C // (_L * _U))
            def _vec(i, mb=mb, ob=ob):
                for u in range(_U):
                    off = i * (_L * _U) + u * _L
                    j = mb[pl.ds(off, _L)]
                    for d in range(3):
                        v = plsc.load_gather(tab_v, [j + _TPAD * d])
                        ob[pl.ds(d * _C + off, _L)] = v

            out_cp[c] = [
                pltpu.async_copy(
                    ob.at[pl.ds(d * _C, _C)],
                    out_hbm.at[pl.ds((b_img * 3 + d) * _PLANE + po + c * _C,
                                     _C)],
                    sout[b])
                for d in range(3)
            ]
            if c + 2 < _NCHUNK:
                in_cp[c + 2] = pltpu.async_copy(
                    mask_hbm.at[pl.ds(base0 + (c + 2) * _C, _C)], mb, sin[b])

        for c in (_NCHUNK - 2, _NCHUNK - 1):
            for cp_ in out_cp[c]:
                cp_.wait()

    return k(mask_flat, mesh_flat, im_pad)


def kernel(mask, mesh_weight, index_map):
    m5 = mask.reshape(_B, _H // 8, 8, _W // 128, 128)
    mask_flat = m5.transpose(0, 1, 3, 2, 4).reshape(-1)
    mesh_flat = jnp.pad(mesh_weight.reshape(-1), (0, _MPAD - mesh_weight.size))
    im_pad = jnp.pad(index_map, (0, _TPAD - index_map.shape[0]))
    oflat = _sc_lookup(mask_flat, mesh_flat, im_pad)
    return oflat.reshape(_B, 3, _H, _W).transpose(0, 2, 3, 1)
